# baseline (speedup 1.0000x reference)
import functools

import jax
import jax.numpy as jnp
from jax import lax
from jax.experimental import pallas as pl
from jax.experimental.pallas import tpu as pltpu
from jax.experimental.pallas import tpu_sc as plsc

D = 64
B = 16384
MARGIN = 1.0
NC = 2
NS = 16
L = 16
NW = NC * NS
PAIRS_W = B // NW
GROUPS_W = PAIRS_W // L
RG = 2 * L
EBLK = 128
NEB = PAIRS_W // EBLK
GPB = EBLK // L
NROW = 1000000
NBLK = NROW // 128
SROW = NROW // 2

_MESH = plsc.VectorSubcoreMesh(core_axis_name="c", subcore_axis_name="s")
_CP = pltpu.CompilerParams(needs_layout_passes=False, use_tc_tiling_on_sc=True)


def _transpose_block(src, dst, ncol):
    iota = lax.iota(jnp.int32, L)
    rots = [(iota + k) & (L - 1) for k in range(L)]

    def tile(t, _):
        tr = (t // (ncol // L)) * L
        tc = (t % (ncol // L)) * L
        idx_r = tr + iota
        for k in range(L):
            rot = rots[k]
            idx_c = tc + rot
            v = plsc.load_gather(src, [idx_r, idx_c])
            row_d = lax.shift_right_logical(idx_c, 1)
            col_d = idx_r + lax.shift_left(rot & 1, 6)
            plsc.store_scatter(dst, [row_d, col_d], v)
        return 0

    lax.fori_loop(0, 4 * (ncol // L), tile, 0)


@functools.partial(
    pl.kernel,
    out_type=jax.ShapeDtypeStruct((SROW, 2 * D), jnp.float32),
    mesh=_MESH,
    scratch_types=[
        pltpu.VMEM((D, 2 * D), jnp.float32),
        pltpu.VMEM((D, 2 * D), jnp.float32),
        pltpu.VMEM((D, 2 * D), jnp.float32),
        pltpu.VMEM((D, 2 * D), jnp.float32),
        pltpu.VMEM((D, D), jnp.float32),
        pltpu.SemaphoreType.DMA,
        pltpu.SemaphoreType.DMA,
        pltpu.SemaphoreType.DMA,
        pltpu.SemaphoreType.DMA,
    ],
    compiler_params=_CP,
)
def _relayout(HT, HTtail, out, inA, inB, outA, outB, tailb, siA, siB, soA,
              soB):
    wid = lax.axis_index("s") * NC + lax.axis_index("c")
    nper = (NBLK + NW - 1) // NW
    nhalf = nper // 2

    def blk_src(j):
        return HT.at[:, pl.ds(pl.multiple_of(j * (2 * D), 2 * D), 2 * D)]

    def blk_dst(j):
        return out.at[pl.ds(pl.multiple_of(j * D, 8), D), :]

    def jat(i):
        return jnp.minimum(wid + i * NW, NBLK - 1)

    pltpu.make_async_copy(blk_src(jat(0)), inA, siA).start()

    def body(i2, _):
        iA = 2 * i2
        iB = iA + 1
        jA = jat(iA)
        jB = jat(iB)

        pltpu.make_async_copy(blk_src(jB), inB, siB).start()
        pltpu.make_async_copy(blk_src(jA), inA, siA).wait()

        @pl.when(i2 > 0)
        def _():
            pltpu.make_async_copy(outA, blk_dst(jA), soA).wait()
        _transpose_block(inA, outA, 2 * D)
        pltpu.make_async_copy(outA, blk_dst(jA), soA).start()
        pltpu.make_async_copy(blk_src(jat(iA + 2)), inA, siA).start()

        pltpu.make_async_copy(blk_src(jB), inB, siB).wait()

        @pl.when(i2 > 0)
        def _():
            pltpu.make_async_copy(outB, blk_dst(jB), soB).wait()
        _transpose_block(inB, outB, 2 * D)
        pltpu.make_async_copy(outB, blk_dst(jB), soB).start()
        return 0

    lax.fori_loop(0, nhalf, body, 0)
    j = jat(nper - 1)
    pltpu.make_async_copy(blk_src(j), inA, siA).wait()
    pltpu.make_async_copy(outA, blk_dst(j), soA).wait()
    _transpose_block(inA, outA, 2 * D)
    pltpu.async_copy(outA, blk_dst(j), soA).wait()
    pltpu.make_async_copy(outB, blk_dst(j), soB).wait()

    @pl.when(wid == 0)
    def _():
        pltpu.sync_copy(HTtail, tailb)
        _transpose_block(tailb, outA, D)
        pltpu.sync_copy(
            outA.at[pl.ds(0, D // 2), :],
            out.at[pl.ds(pl.multiple_of(NBLK * D, 8), D // 2), :])


def _rsqrt_nr(x):
    i = plsc.bitcast(x, jnp.int32)
    i = jnp.int32(0x5F3759DF) - lax.shift_right_logical(i, 1)
    y = plsc.bitcast(i, jnp.float32)
    for _ in range(3):
        y = y * (1.5 - 0.5 * x * y * y)
    return y


def _group_dists(rows, hv, tv, e_ref, col0):
    iota = lax.iota(jnp.int32, L)
    hpos = 2 * iota
    tpos = hpos + 1
    hhalf = lax.shift_left(hv & 1, 6)
    thalf = lax.shift_left(tv & 1, 6)
    z = jnp.zeros((L,), jnp.float32)

    @plsc.parallel_loop(0, D, 1, unroll=8, carry=(z, z, z, z, z, z))
    def acc(d, c):
        hh, tt, ee, he, ht_, et = c
        ds = jnp.full((L,), d, jnp.int32)
        h = plsc.load_gather(rows, [hpos, hhalf + ds])
        t = plsc.load_gather(rows, [tpos, thalf + ds])
        e = plsc.load_gather(e_ref, [ds, col0 + iota])
        return (hh + h * h, tt + t * t, ee + e * e,
                he + h * e, ht_ + h * t, et + e * t)

    hh, tt, ee, he, ht_, et = acc
    rh = _rsqrt_nr(jnp.maximum(hh, 1e-24))
    rt = _rsqrt_nr(jnp.maximum(tt, 1e-24))
    d2 = 2.0 + ee + 2.0 * (he * rh - ht_ * (rh * rt) - et * rt)
    d2 = jnp.maximum(d2, 0.0)
    return d2 * _rsqrt_nr(jnp.maximum(d2, 1e-24))


@functools.partial(
    pl.kernel,
    out_type=jax.ShapeDtypeStruct((NW * L,), jnp.float32),
    mesh=_MESH,
    scratch_types=[
        pltpu.VMEM((2 * PAIRS_W,), jnp.int32),
        pltpu.VMEM((2 * PAIRS_W,), jnp.int32),
        pltpu.VMEM((2 * L,), jnp.int32),
        pltpu.VMEM((2 * L,), jnp.int32),
        pltpu.VMEM((D, EBLK), jnp.float32),
        pltpu.VMEM((D, EBLK), jnp.float32),
        pltpu.VMEM((D, EBLK), jnp.float32),
        pltpu.VMEM((D, EBLK), jnp.float32),
        pltpu.VMEM((D, EBLK), jnp.float32),
        pltpu.VMEM((RG, 2 * D), jnp.float32),
        pltpu.VMEM((RG, 2 * D), jnp.float32),
        pltpu.VMEM((L,), jnp.float32),
        pltpu.SemaphoreType.DMA,
        pltpu.SemaphoreType.DMA,
    ],
    compiler_params=_CP,
)
def _sc_loss(H2, ET, ht_flat, out, idx_own, idx_sup, idx0, idx0_sup,
             e0, e_0, e_1, e_2, e_3, rowsA, rowsB, vec, semA, semB):
    wid = lax.axis_index("s") * NC + lax.axis_index("c")
    pbase = wid * PAIRS_W
    iota = lax.iota(jnp.int32, L)
    eblks = (e_0, e_1, e_2, e_3)

    pltpu.sync_copy(
        ht_flat.at[pl.ds(pl.multiple_of(pbase * 2, 1024), 2 * PAIRS_W)],
        idx_own)
    pltpu.sync_copy(ht_flat.at[pl.ds(0, 2 * L)], idx0)
    pltpu.sync_copy(ET.at[:, pl.ds(0, EBLK)], e0)
    for k in range(NEB):
        pltpu.sync_copy(
            ET.at[:, pl.ds(pl.multiple_of(pbase + k * EBLK, EBLK), EBLK)],
            eblks[k])

    @plsc.parallel_loop(0, 2 * PAIRS_W, L, unroll=4)
    def _shift(i):
        idx_sup[pl.ds(i, L)] = lax.shift_right_logical(idx_own[pl.ds(i, L)], 1)

    @plsc.parallel_loop(0, 2 * L, L)
    def _shift0(i):
        idx0_sup[pl.ds(i, L)] = lax.shift_right_logical(idx0[pl.ds(i, L)], 1)

    hv0 = plsc.load_gather(idx0, [2 * iota])
    tv0 = plsc.load_gather(idx0, [2 * iota + 1])
    pltpu.async_copy(H2.at[idx0_sup], rowsA, semA).wait()
    d0vec = _group_dists(rowsA, hv0, tv0, e0, 0)
    d0 = d0vec[0]

    def _idx_vecs(g):
        hv = plsc.load_gather(idx_own, [g * RG + 2 * iota])
        tv = plsc.load_gather(idx_own, [g * RG + 2 * iota + 1])
        return hv, tv

    def _issue(g, rows, sem):
        pltpu.make_async_copy(
            H2.at[idx_sup.at[pl.ds(g * RG, RG)]], rows, sem).start()

    def _wait(rows, sem):
        pltpu.make_async_copy(H2.at[pl.ds(0, RG)], rows, sem).wait()

    def _hinge(g, dg):
        rel = jnp.maximum(0.0, (MARGIN + d0) - dg)
        pid = pbase + g * L + iota
        return jnp.where(pid == 0, 0.0, rel)

    _issue(0, rowsA, semA)
    hvA, tvA = _idx_vecs(0)

    s_total = jnp.zeros((L,), jnp.float32)
    for k in range(NEB):
        e_ref = eblks[k]

        def chunk(i, carry):
            s_acc, hvA, tvA = carry
            gA = k * GPB + 2 * i
            gB = gA + 1
            _wait(rowsA, semA)
            _issue(gB, rowsB, semB)
            hvB, tvB = _idx_vecs(gB)
            dA = _group_dists(rowsA, hvA, tvA, e_ref, (2 * i) * L)
            s_acc = s_acc + _hinge(gA, dA)
            _wait(rowsB, semB)
            gN = jnp.minimum(gA + 2, GROUPS_W - 1)
            _issue(gN, rowsA, semA)
            hvN, tvN = _idx_vecs(gN)
            dB = _group_dists(rowsB, hvB, tvB, e_ref, (2 * i + 1) * L)
            s_acc = s_acc + _hinge(gB, dB)
            return s_acc, hvN, tvN

        s_total, hvA, tvA = lax.fori_loop(
            0, GPB // 2, chunk, (s_total, hvA, tvA))

    _wait(rowsA, semA)

    vec[...] = s_total
    pltpu.sync_copy(vec, out.at[pl.ds(pl.multiple_of(wid * L, L), L)])


def _finish_body(p_ref, o_ref):
    o_ref[0, 0] = jnp.sum(p_ref[...]) * (1.0 / (B - 1))


_finish = pl.pallas_call(
    _finish_body,
    out_shape=jax.ShapeDtypeStruct((1, 1), jnp.float32),
    out_specs=pl.BlockSpec(memory_space=pltpu.SMEM),
)


def kernel(H, E, ht, labels, queries, y):
    H2 = _relayout(H.T, H[NBLK * 2 * D:].T)
    partials = _sc_loss(H2, E.T, ht.reshape(-1))
    return _finish(partials.reshape(4, 128))[0, 0]


# speedup vs baseline: 1.8835x; 1.8835x over previous
import functools

import jax
import jax.numpy as jnp
from jax import lax
from jax.experimental import pallas as pl
from jax.experimental.pallas import tpu as pltpu
from jax.experimental.pallas import tpu_sc as plsc

D = 64
B = 16384
MARGIN = 1.0
NC = 2
NS = 16
L = 16
NW = NC * NS
PAIRS_W = B // NW
GROUPS_W = PAIRS_W // L
RG = 2 * L
EBLK = 128
NEB = PAIRS_W // EBLK
GPB = EBLK // L
NROW = 1000000
NBLK = NROW // 128
SROW = NROW // 2

_MESH = plsc.VectorSubcoreMesh(core_axis_name="c", subcore_axis_name="s")
_CP = pltpu.CompilerParams(needs_layout_passes=False, use_tc_tiling_on_sc=True)


def _transpose_block(src, dst, ncol):
    iota = lax.iota(jnp.int32, L)
    rots = [(iota + k) & (L - 1) for k in range(L)]

    @plsc.parallel_loop(0, 4 * (ncol // L), 1, unroll=4)
    def tile(t):
        tr = (t // (ncol // L)) * L
        tc = (t % (ncol // L)) * L
        idx_r = tr + iota
        for k in range(L):
            rot = rots[k]
            idx_c = tc + rot
            v = plsc.load_gather(src, [idx_r, idx_c])
            row_d = lax.shift_right_logical(idx_c, 1)
            col_d = idx_r + lax.shift_left(rot & 1, 6)
            plsc.store_scatter(dst, [row_d, col_d], v)


@functools.partial(
    pl.kernel,
    out_type=jax.ShapeDtypeStruct((SROW, 2 * D), jnp.float32),
    mesh=_MESH,
    scratch_types=[
        pltpu.VMEM((D, 2 * D), jnp.float32),
        pltpu.VMEM((D, 2 * D), jnp.float32),
        pltpu.VMEM((D, 2 * D), jnp.float32),
        pltpu.VMEM((D, 2 * D), jnp.float32),
        pltpu.VMEM((D, D), jnp.float32),
        pltpu.SemaphoreType.DMA,
        pltpu.SemaphoreType.DMA,
        pltpu.SemaphoreType.DMA,
        pltpu.SemaphoreType.DMA,
    ],
    compiler_params=_CP,
)
def _relayout(HT, HTtail, out, inA, inB, outA, outB, tailb, siA, siB, soA,
              soB):
    wid = lax.axis_index("s") * NC + lax.axis_index("c")
    nper = (NBLK + NW - 1) // NW
    nhalf = nper // 2

    def blk_src(j):
        return HT.at[:, pl.ds(pl.multiple_of(j * (2 * D), 2 * D), 2 * D)]

    def blk_dst(j):
        return out.at[pl.ds(pl.multiple_of(j * D, 8), D), :]

    def jat(i):
        return jnp.minimum(wid + i * NW, NBLK - 1)

    pltpu.make_async_copy(blk_src(jat(0)), inA, siA).start()

    def body(i2, _):
        iA = 2 * i2
        iB = iA + 1
        jA = jat(iA)
        jB = jat(iB)

        pltpu.make_async_copy(blk_src(jB), inB, siB).start()
        pltpu.make_async_copy(blk_src(jA), inA, siA).wait()

        @pl.when(i2 > 0)
        def _():
            pltpu.make_async_copy(outA, blk_dst(jA), soA).wait()
        _transpose_block(inA, outA, 2 * D)
        pltpu.make_async_copy(outA, blk_dst(jA), soA).start()
        pltpu.make_async_copy(blk_src(jat(iA + 2)), inA, siA).start()

        pltpu.make_async_copy(blk_src(jB), inB, siB).wait()

        @pl.when(i2 > 0)
        def _():
            pltpu.make_async_copy(outB, blk_dst(jB), soB).wait()
        _transpose_block(inB, outB, 2 * D)
        pltpu.make_async_copy(outB, blk_dst(jB), soB).start()
        return 0

    lax.fori_loop(0, nhalf, body, 0)
    j = jat(nper - 1)
    pltpu.make_async_copy(blk_src(j), inA, siA).wait()
    pltpu.make_async_copy(outA, blk_dst(j), soA).wait()
    _transpose_block(inA, outA, 2 * D)
    pltpu.async_copy(outA, blk_dst(j), soA).wait()
    pltpu.make_async_copy(outB, blk_dst(j), soB).wait()

    @pl.when(wid == 0)
    def _():
        pltpu.sync_copy(HTtail, tailb)
        _transpose_block(tailb, outA, D)
        pltpu.sync_copy(
            outA.at[pl.ds(0, D // 2), :],
            out.at[pl.ds(pl.multiple_of(NBLK * D, 8), D // 2), :])


def _rsqrt_nr(x):
    i = plsc.bitcast(x, jnp.int32)
    i = jnp.int32(0x5F3759DF) - lax.shift_right_logical(i, 1)
    y = plsc.bitcast(i, jnp.float32)
    for _ in range(3):
        y = y * (1.5 - 0.5 * x * y * y)
    return y


def _group_dists(rows, hv, tv, e_ref, col0):
    iota = lax.iota(jnp.int32, L)
    hpos = 2 * iota
    tpos = hpos + 1
    hhalf = lax.shift_left(hv & 1, 6)
    thalf = lax.shift_left(tv & 1, 6)
    z = jnp.zeros((L,), jnp.float32)

    @plsc.parallel_loop(0, D, 1, unroll=8, carry=(z, z, z, z, z, z))
    def acc(d, c):
        hh, tt, ee, he, ht_, et = c
        ds = jnp.full((L,), d, jnp.int32)
        h = plsc.load_gather(rows, [hpos, hhalf + ds])
        t = plsc.load_gather(rows, [tpos, thalf + ds])
        e = plsc.load_gather(e_ref, [ds, col0 + iota])
        return (hh + h * h, tt + t * t, ee + e * e,
                he + h * e, ht_ + h * t, et + e * t)

    hh, tt, ee, he, ht_, et = acc
    rh = _rsqrt_nr(jnp.maximum(hh, 1e-24))
    rt = _rsqrt_nr(jnp.maximum(tt, 1e-24))
    d2 = 2.0 + ee + 2.0 * (he * rh - ht_ * (rh * rt) - et * rt)
    d2 = jnp.maximum(d2, 0.0)
    return d2 * _rsqrt_nr(jnp.maximum(d2, 1e-24))


@functools.partial(
    pl.kernel,
    out_type=jax.ShapeDtypeStruct((NW * L,), jnp.float32),
    mesh=_MESH,
    scratch_types=[
        pltpu.VMEM((2 * PAIRS_W,), jnp.int32),
        pltpu.VMEM((2 * PAIRS_W,), jnp.int32),
        pltpu.VMEM((2 * L,), jnp.int32),
        pltpu.VMEM((2 * L,), jnp.int32),
        pltpu.VMEM((D, EBLK), jnp.float32),
        pltpu.VMEM((D, EBLK), jnp.float32),
        pltpu.VMEM((D, EBLK), jnp.float32),
        pltpu.VMEM((D, EBLK), jnp.float32),
        pltpu.VMEM((D, EBLK), jnp.float32),
        pltpu.VMEM((RG, 2 * D), jnp.float32),
        pltpu.VMEM((RG, 2 * D), jnp.float32),
        pltpu.VMEM((L,), jnp.float32),
        pltpu.SemaphoreType.DMA,
        pltpu.SemaphoreType.DMA,
    ],
    compiler_params=_CP,
)
def _sc_loss(H2, ET, ht_flat, out, idx_own, idx_sup, idx0, idx0_sup,
             e0, e_0, e_1, e_2, e_3, rowsA, rowsB, vec, semA, semB):
    wid = lax.axis_index("s") * NC + lax.axis_index("c")
    pbase = wid * PAIRS_W
    iota = lax.iota(jnp.int32, L)
    eblks = (e_0, e_1, e_2, e_3)

    pltpu.sync_copy(
        ht_flat.at[pl.ds(pl.multiple_of(pbase * 2, 1024), 2 * PAIRS_W)],
        idx_own)
    pltpu.sync_copy(ht_flat.at[pl.ds(0, 2 * L)], idx0)
    pltpu.sync_copy(ET.at[:, pl.ds(0, EBLK)], e0)
    for k in range(NEB):
        pltpu.sync_copy(
            ET.at[:, pl.ds(pl.multiple_of(pbase + k * EBLK, EBLK), EBLK)],
            eblks[k])

    @plsc.parallel_loop(0, 2 * PAIRS_W, L, unroll=4)
    def _shift(i):
        idx_sup[pl.ds(i, L)] = lax.shift_right_logical(idx_own[pl.ds(i, L)], 1)

    @plsc.parallel_loop(0, 2 * L, L)
    def _shift0(i):
        idx0_sup[pl.ds(i, L)] = lax.shift_right_logical(idx0[pl.ds(i, L)], 1)

    hv0 = plsc.load_gather(idx0, [2 * iota])
    tv0 = plsc.load_gather(idx0, [2 * iota + 1])
    pltpu.async_copy(H2.at[idx0_sup], rowsA, semA).wait()
    d0vec = _group_dists(rowsA, hv0, tv0, e0, 0)
    d0 = d0vec[0]

    def _idx_vecs(g):
        hv = plsc.load_gather(idx_own, [g * RG + 2 * iota])
        tv = plsc.load_gather(idx_own, [g * RG + 2 * iota + 1])
        return hv, tv

    def _issue(g, rows, sem):
        pltpu.make_async_copy(
            H2.at[idx_sup.at[pl.ds(g * RG, RG)]], rows, sem).start()

    def _wait(rows, sem):
        pltpu.make_async_copy(H2.at[pl.ds(0, RG)], rows, sem).wait()

    def _hinge(g, dg):
        rel = jnp.maximum(0.0, (MARGIN + d0) - dg)
        pid = pbase + g * L + iota
        return jnp.where(pid == 0, 0.0, rel)

    _issue(0, rowsA, semA)
    hvA, tvA = _idx_vecs(0)

    s_total = jnp.zeros((L,), jnp.float32)
    for k in range(NEB):
        e_ref = eblks[k]

        def chunk(i, carry):
            s_acc, hvA, tvA = carry
            gA = k * GPB + 2 * i
            gB = gA + 1
            _wait(rowsA, semA)
            _issue(gB, rowsB, semB)
            hvB, tvB = _idx_vecs(gB)
            dA = _group_dists(rowsA, hvA, tvA, e_ref, (2 * i) * L)
            s_acc = s_acc + _hinge(gA, dA)
            _wait(rowsB, semB)
            gN = jnp.minimum(gA + 2, GROUPS_W - 1)
            _issue(gN, rowsA, semA)
            hvN, tvN = _idx_vecs(gN)
            dB = _group_dists(rowsB, hvB, tvB, e_ref, (2 * i + 1) * L)
            s_acc = s_acc + _hinge(gB, dB)
            return s_acc, hvN, tvN

        s_total, hvA, tvA = lax.fori_loop(
            0, GPB // 2, chunk, (s_total, hvA, tvA))

    _wait(rowsA, semA)

    vec[...] = s_total
    pltpu.sync_copy(vec, out.at[pl.ds(pl.multiple_of(wid * L, L), L)])


def _finish_body(p_ref, o_ref):
    o_ref[0, 0] = jnp.sum(p_ref[...]) * (1.0 / (B - 1))


_finish = pl.pallas_call(
    _finish_body,
    out_shape=jax.ShapeDtypeStruct((1, 1), jnp.float32),
    out_specs=pl.BlockSpec(memory_space=pltpu.SMEM),
)


def kernel(H, E, ht, labels, queries, y):
    H2 = _relayout(H.T, H[NBLK * 2 * D:].T)
    partials = _sc_loss(H2, E.T, ht.reshape(-1))
    return _finish(partials.reshape(4, 128))[0, 0]
